# baseline (speedup 1.0000x reference)
import functools

import jax
import jax.numpy as jnp
from jax import lax
from jax.experimental import pallas as pl
from jax.experimental.pallas import tpu as pltpu
from jax.experimental.pallas import tpu_sc as plsc

N = 1024
F_IN = 128
F = 64

_NC = 2
_NS = 16
_NW = _NC * _NS
_L = 16
_ROWS_PER_TILE = N // _NW


def _prep_body(x_ref, w_ref, b_ref, a2_ref, h_ref, s2_ref):
    h = jnp.dot(x_ref[...], w_ref[...], preferred_element_type=jnp.float32)
    h = h + b_ref[...]
    h_ref[...] = h
    s2_ref[...] = jnp.dot(h, a2_ref[...], preferred_element_type=jnp.float32)


_tc_prep = pl.pallas_call(
    _prep_body,
    out_shape=[
        jax.ShapeDtypeStruct((N, F), jnp.float32),
        jax.ShapeDtypeStruct((N, 1), jnp.float32),
    ],
)


_sc_mesh = plsc.VectorSubcoreMesh(core_axis_name="c", subcore_axis_name="s")


_CHUNK = N * N // _NW


@functools.partial(
    pl.kernel,
    mesh=_sc_mesh,
    out_type=jax.ShapeDtypeStruct((N * N,), jnp.float32),
    compiler_params=pltpu.CompilerParams(needs_layout_passes=False),
    scratch_types=[
        pltpu.VMEM((_CHUNK,), jnp.int32),
        pltpu.VMEM((_CHUNK,), jnp.float32),
        pltpu.VMEM((N,), jnp.float32),
    ],
)
def _sc_gather(adj_hbm, s2_hbm, out_hbm, adj_v, g_v, s2_v):
    wid = lax.axis_index("s") * _NC + lax.axis_index("c")
    base = wid * _CHUNK
    pltpu.sync_copy(s2_hbm, s2_v)
    pltpu.sync_copy(adj_hbm.at[pl.ds(base, _CHUNK)], adj_v)

    @plsc.parallel_loop(0, _CHUNK, _L, unroll=8)
    def _body(i):
        idx = adj_v[pl.ds(i, _L)]
        g_v[pl.ds(i, _L)] = plsc.load_gather(s2_v, [idx])

    pltpu.sync_copy(g_v, out_hbm.at[pl.ds(base, _CHUNK)])


_BLK = 256


def _attn_body(g_ref, hr_ref, hf_ref, a1_ref, ab_ref, o_ref):
    s1 = jnp.dot(hr_ref[...], a1_ref[...], preferred_element_type=jnp.float32)
    e = g_ref[...] + s1 + ab_ref[...]
    e = jnp.where(e >= 0.0, e, 0.2 * e)
    m = jnp.max(e, axis=1, keepdims=True)
    p = jnp.exp(e - m)
    s = jnp.sum(p, axis=1, keepdims=True)
    o_ref[...] = jnp.dot(p / s, hf_ref[...], preferred_element_type=jnp.float32)


_tc_attn = pl.pallas_call(
    _attn_body,
    grid=(N // _BLK,),
    in_specs=[
        pl.BlockSpec((_BLK, N), lambda i: (i, 0)),
        pl.BlockSpec((_BLK, F), lambda i: (i, 0)),
        pl.BlockSpec((N, F), lambda i: (0, 0)),
        pl.BlockSpec((F, 1), lambda i: (0, 0)),
        pl.BlockSpec((1, 1), lambda i: (0, 0)),
    ],
    out_specs=pl.BlockSpec((_BLK, F), lambda i: (i, 0)),
    out_shape=jax.ShapeDtypeStruct((N, F), jnp.float32),
)


def kernel(X, adj, W_w, W_b, a_w, a_b):
    adj32 = adj.astype(jnp.int32)
    a1 = a_w[:F]
    a2 = a_w[F:]
    h, s2 = _tc_prep(X, W_w, W_b.reshape(1, F), a2)
    g = _sc_gather(adj32.reshape(N * N), s2.reshape(N))
    return _tc_attn(g.reshape(N, N), h, h, a1, a_b.reshape(1, 1))


# speedup vs baseline: 1.2499x; 1.2499x over previous
import functools

import jax
import jax.numpy as jnp
from jax import lax
from jax.experimental import pallas as pl
from jax.experimental.pallas import tpu as pltpu
from jax.experimental.pallas import tpu_sc as plsc

N = 1024
F_IN = 128
F = 64

_NC = 2
_NS = 16
_NW = _NC * _NS
_L = 16
_ROWS_PER_TILE = N // _NW


def _prep_body(x_ref, w_ref, b_ref, a2_ref, h_ref, s2_ref):
    h = jnp.dot(x_ref[...], w_ref[...], preferred_element_type=jnp.float32)
    h = h + b_ref[...]
    h_ref[...] = h
    s2_ref[...] = jnp.dot(h, a2_ref[...], preferred_element_type=jnp.float32)


_tc_prep = pl.pallas_call(
    _prep_body,
    out_shape=[
        jax.ShapeDtypeStruct((N, F), jnp.float32),
        jax.ShapeDtypeStruct((N, 1), jnp.float32),
    ],
)


_sc_mesh = plsc.VectorSubcoreMesh(core_axis_name="c", subcore_axis_name="s")


@functools.partial(
    pl.kernel,
    mesh=_sc_mesh,
    out_type=jax.ShapeDtypeStruct((N, N), jnp.float32),
    compiler_params=pltpu.CompilerParams(needs_layout_passes=False),
    scratch_types=[
        pltpu.VMEM((_ROWS_PER_TILE, N), jnp.int32),
        pltpu.VMEM((_ROWS_PER_TILE, N), jnp.float32),
        pltpu.VMEM((N,), jnp.float32),
    ],
)
def _sc_gather(adj_hbm, s2_hbm, out_hbm, adj_v, g_v, s2_v):
    wid = lax.axis_index("s") * _NC + lax.axis_index("c")
    base = wid * _ROWS_PER_TILE
    pltpu.sync_copy(s2_hbm, s2_v)
    pltpu.sync_copy(adj_hbm.at[pl.ds(base, _ROWS_PER_TILE)], adj_v)

    def _row(r, carry):
        @plsc.parallel_loop(0, N, _L, unroll=8)
        def _col(j):
            idx = adj_v[r, pl.ds(j, _L)]
            g_v[r, pl.ds(j, _L)] = plsc.load_gather(s2_v, [idx])

        return carry

    lax.fori_loop(0, _ROWS_PER_TILE, _row, 0)
    pltpu.sync_copy(g_v, out_hbm.at[pl.ds(base, _ROWS_PER_TILE)])


_BLK = 256


def _attn_body(g_ref, hr_ref, hf_ref, a1_ref, ab_ref, o_ref):
    s1 = jnp.dot(hr_ref[...], a1_ref[...], preferred_element_type=jnp.float32)
    e = g_ref[...] + s1 + ab_ref[...]
    e = jnp.where(e >= 0.0, e, 0.2 * e)
    m = jnp.max(e, axis=1, keepdims=True)
    p = jnp.exp(e - m)
    s = jnp.sum(p, axis=1, keepdims=True)
    o_ref[...] = jnp.dot(p / s, hf_ref[...], preferred_element_type=jnp.float32)


_tc_attn = pl.pallas_call(
    _attn_body,
    grid=(N // _BLK,),
    in_specs=[
        pl.BlockSpec((_BLK, N), lambda i: (i, 0)),
        pl.BlockSpec((_BLK, F), lambda i: (i, 0)),
        pl.BlockSpec((N, F), lambda i: (0, 0)),
        pl.BlockSpec((F, 1), lambda i: (0, 0)),
        pl.BlockSpec((1, 1), lambda i: (0, 0)),
    ],
    out_specs=pl.BlockSpec((_BLK, F), lambda i: (i, 0)),
    out_shape=jax.ShapeDtypeStruct((N, F), jnp.float32),
)


def kernel(X, adj, W_w, W_b, a_w, a_b):
    adj32 = adj.astype(jnp.int32)
    a1 = a_w[:F]
    a2 = a_w[F:]
    h, s2 = _tc_prep(X, W_w, W_b.reshape(1, F), a2)
    g = _sc_gather(adj32, s2.reshape(N))
    return _tc_attn(g, h, h, a1, a_b.reshape(1, 1))
